# baseline (speedup 1.0000x reference)
import functools

import jax
import jax.numpy as jnp
from jax import lax
from jax.experimental import pallas as pl
from jax.experimental.pallas import tpu as pltpu
from jax.experimental.pallas import tpu_sc as plsc

CAP = 100000
SEQ = 20
HID = 64
K = 32
L = 16
NC, NS = 2, 16
NW = NC * NS
GROUPS = CAP // L
CG = 20
NCHUNK = 10
BG = 10
ROWS_PER_CHUNK = CG * L
MAXG_W = 196
NEG_INF = float("-inf")


def _iota16():
    return lax.iota(jnp.int32, L)


def _splat_f(x):
    return jnp.full((L,), x, dtype=jnp.float32)


def _splat_i(x):
    return jnp.full((L,), x, dtype=jnp.int32)


def _rsqrt16(x):
    i = plsc.bitcast(x, jnp.int32)
    i = jnp.int32(0x5F3759DF) - (i >> 1)
    r = plsc.bitcast(i, jnp.float32)
    for _ in range(3):
        r = r * (1.5 - 0.5 * x * r * r)
    return r


_MESH = plsc.VectorSubcoreMesh(core_axis_name="c", subcore_axis_name="s")
_PARAMS = pltpu.CompilerParams(needs_layout_passes=False)


@functools.partial(
    pl.kernel,
    out_type=(
        jax.ShapeDtypeStruct((NW * K,), jnp.float32),
        jax.ShapeDtypeStruct((NW * K,), jnp.int32),
    ),
    mesh=_MESH,
    scratch_types=[
        pltpu.VMEM((ROWS_PER_CHUNK, HID), jnp.float32),
        pltpu.VMEM((MAXG_W * L,), jnp.float32),
        pltpu.VMEM((HID * L,), jnp.float32),
        pltpu.VMEM((K,), jnp.float32),
        pltpu.VMEM((K,), jnp.int32),
    ],
    compiler_params=_PARAMS,
)
def _partial_topk(emb_hbm, q_hbm, cval_hbm, cidx_hbm, buf, sims, qv, cv, ci):
    wid = lax.axis_index("s") * NC + lax.axis_index("c")
    start_g = (wid * GROUPS) // NW
    n_g = ((wid + 1) * GROUPS) // NW - start_g
    iota = _iota16()

    pltpu.sync_copy(q_hbm, qv)

    def chunk_body(c, _):
        loc = jnp.minimum(c * CG, n_g - CG)
        row0 = (start_g + loc) * L
        pltpu.sync_copy(emb_hbm.at[pl.ds(row0, ROWS_PER_CHUNK)], buf)

        def block_body(b, _):
            g0 = b * BG
            rows = [(g0 + s) * L + iota for s in range(BG)]
            acc = [_splat_f(0.0) for _ in range(BG)]
            nacc = [_splat_f(0.0) for _ in range(BG)]
            for h in range(HID):
                col = _splat_i(h)
                qh = qv[pl.ds(h * L, L)]
                for s in range(BG):
                    v = plsc.load_gather(buf, [rows[s], col])
                    acc[s] = acc[s] + v * qh
                    nacc[s] = nacc[s] + v * v
            for s in range(BG):
                en = jnp.maximum(nacc[s] * _rsqrt16(nacc[s]), 1e-8)
                sim = acc[s] / en
                sims[pl.ds((loc + g0 + s) * L, L)] = sim
            return 0

        lax.fori_loop(0, CG // BG, block_body, 0)
        return 0

    lax.fori_loop(0, NCHUNK, chunk_body, 0)

    mask0 = iota == 0
    base_elem = start_g * L

    def select_body(j, _):
        def scan_body(g, ma):
            m, a = ma
            v = sims[pl.ds(g * L, L)]
            idxv = _splat_i(base_elem + g * L) + iota
            upd = v > m
            return jnp.where(upd, v, m), jnp.where(upd, idxv, a)

        m, a = lax.fori_loop(
            0, n_g, scan_body, (_splat_f(NEG_INF), _splat_i(0))
        )
        mx = jnp.max(m)
        eq = m == _splat_f(mx)
        pos = jnp.min(jnp.where(eq, a, jnp.int32(2**30)))
        jv = _splat_i(j)
        plsc.store_scatter(cv, [jv], _splat_f(mx), mask=mask0)
        plsc.store_scatter(ci, [jv], _splat_i(pos), mask=mask0)
        plsc.store_scatter(
            sims, [_splat_i(pos - base_elem)], _splat_f(NEG_INF), mask=mask0
        )
        return 0

    lax.fori_loop(0, K, select_body, 0)

    pltpu.sync_copy(cv, cval_hbm.at[pl.ds(wid * K, K)])
    pltpu.sync_copy(ci, cidx_hbm.at[pl.ds(wid * K, K)])


@functools.partial(
    pl.kernel,
    out_type=(
        jax.ShapeDtypeStruct((K,), jnp.float32),
        jax.ShapeDtypeStruct((K,), jnp.int32),
    ),
    mesh=_MESH,
    scratch_types=[
        pltpu.VMEM((NW * K,), jnp.float32),
        pltpu.VMEM((NW * K,), jnp.int32),
        pltpu.VMEM((K,), jnp.float32),
        pltpu.VMEM((K,), jnp.int32),
        pltpu.VMEM((HID,), jnp.float32),
    ],
    compiler_params=_PARAMS,
)
def _merge(cval_hbm, cidx_hbm, q_hbm, score_hbm, idx_hbm,
           cvv, cii, selv, seli, qv):
    wid = lax.axis_index("s") * NC + lax.axis_index("c")
    iota = _iota16()
    mask0 = iota == 0

    @pl.when(wid == 0)
    def _():
        pltpu.sync_copy(cval_hbm, cvv)
        pltpu.sync_copy(cidx_hbm, cii)
        pltpu.sync_copy(q_hbm, qv)

        qsq = _splat_f(0.0)
        for t in range(HID // L):
            vq = qv[pl.ds(t * L, L)]
            qsq = qsq + vq * vq
        sv = _splat_f(jnp.sum(qsq))
        qn = jnp.maximum(sv * _rsqrt16(sv), 1e-8)

        def select_body(j, _):
            def scan_body(g, mae):
                m, a, e = mae
                v = cvv[pl.ds(g * L, L)]
                vi = cii[pl.ds(g * L, L)]
                idxv = _splat_i(g * L) + iota
                upd = v > m
                return (jnp.where(upd, v, m), jnp.where(upd, idxv, a),
                        jnp.where(upd, vi, e))

            m, a, e = lax.fori_loop(
                0, NW * K // L, scan_body,
                (_splat_f(NEG_INF), _splat_i(0), _splat_i(0)),
            )
            mx = jnp.max(m)
            eq = m == _splat_f(mx)
            pos = jnp.min(jnp.where(eq, a, jnp.int32(2**30)))
            posv = _splat_i(pos)
            epi_idx = jnp.min(jnp.where(a == posv, e, jnp.int32(2**30)))
            jv = _splat_i(j)
            plsc.store_scatter(selv, [jv], _splat_f(mx) / qn, mask=mask0)
            plsc.store_scatter(seli, [jv], _splat_i(epi_idx), mask=mask0)
            plsc.store_scatter(cvv, [posv], _splat_f(NEG_INF), mask=mask0)
            return 0

        lax.fori_loop(0, K, select_body, 0)

        pltpu.sync_copy(selv, score_hbm)
        pltpu.sync_copy(seli, idx_hbm)


def _gather_body(idx_ref, epi_ref, out_ref):
    out_ref[...] = epi_ref[...]


def _gather_tc(idx, episodes):
    grid_spec = pltpu.PrefetchScalarGridSpec(
        num_scalar_prefetch=1,
        grid=(K,),
        in_specs=[
            pl.BlockSpec((1, SEQ, HID), lambda i, idx_ref: (idx_ref[i], 0, 0))
        ],
        out_specs=pl.BlockSpec((1, SEQ, HID), lambda i, idx_ref: (i, 0, 0)),
    )
    return pl.pallas_call(
        _gather_body,
        grid_spec=grid_spec,
        out_shape=jax.ShapeDtypeStruct((K, SEQ, HID), jnp.float32),
    )(idx, episodes)


def kernel(query, k, episodes, episode_embeddings):
    if query.ndim == 1:
        query = query[None, :]
    q0 = query[0]
    qb = jnp.repeat(q0, L)
    cval, cidx = _partial_topk(episode_embeddings, qb)
    scores, top_idx = _merge(cval, cidx, q0)
    retr = _gather_tc(top_idx, episodes)
    scores = scores + jnp.asarray(k - k, dtype=scores.dtype)
    return retr, scores


# speedup vs baseline: 8.9663x; 8.9663x over previous
import functools

import jax
import jax.numpy as jnp
from jax import lax
from jax.experimental import pallas as pl
from jax.experimental.pallas import tpu as pltpu
from jax.experimental.pallas import tpu_sc as plsc

CAP = 100000
SEQ = 20
HID = 64
K = 32
L = 16
NC, NS = 2, 16
NW = NC * NS
TILE = 128
NT_FULL = CAP // TILE
REM = CAP - NT_FULL * TILE
CT = 5
COLS_PER_CHUNK = CT * TILE
NCHUNK = 5
BG = 10
GP_CHUNK = COLS_PER_CHUNK // L
MAXG_W = 25 * (TILE // L) + 2
NEG_INF = float("-inf")


def _iota16():
    return lax.iota(jnp.int32, L)


def _splat_f(x):
    return jnp.full((L,), x, dtype=jnp.float32)


def _splat_i(x):
    return jnp.full((L,), x, dtype=jnp.int32)


def _rsqrt16(x):
    i = plsc.bitcast(x, jnp.int32)
    i = jnp.int32(0x5F3759DF) - (i >> 1)
    r = plsc.bitcast(i, jnp.float32)
    for _ in range(3):
        r = r * (1.5 - 0.5 * x * r * r)
    return r


_MESH = plsc.VectorSubcoreMesh(core_axis_name="c", subcore_axis_name="s")
_PARAMS = pltpu.CompilerParams(needs_layout_passes=False)


@functools.partial(
    pl.kernel,
    out_type=(
        jax.ShapeDtypeStruct((NW * K,), jnp.float32),
        jax.ShapeDtypeStruct((NW * K,), jnp.int32),
    ),
    mesh=_MESH,
    scratch_types=[
        pltpu.VMEM((HID, COLS_PER_CHUNK), jnp.float32),
        pltpu.VMEM((HID, REM), jnp.float32),
        pltpu.VMEM((MAXG_W * L,), jnp.float32),
        pltpu.VMEM((HID * L,), jnp.float32),
        pltpu.VMEM((K,), jnp.float32),
        pltpu.VMEM((K,), jnp.int32),
    ],
    compiler_params=_PARAMS,
)
def _partial_topk(emb_hbm, tail_hbm, q_hbm, cval_hbm, cidx_hbm,
                  buf, tbuf, sims, qv, cv, ci):
    wid = lax.axis_index("s") * NC + lax.axis_index("c")
    t0 = (wid * NT_FULL) // NW
    n_t = ((wid + 1) * NT_FULL) // NW - t0
    iota = _iota16()

    pltpu.sync_copy(q_hbm, qv)

    def _dot_groups(src, col_base, sim_base, n):
        acc = [_splat_f(0.0) for _ in range(n)]
        nacc = [_splat_f(0.0) for _ in range(n)]
        for h in range(HID):
            qh = qv[pl.ds(h * L, L)]
            for s in range(n):
                v = src[h, pl.ds(col_base + s * L, L)]
                acc[s] = acc[s] + v * qh
                nacc[s] = nacc[s] + v * v
        for s in range(n):
            en = jnp.maximum(nacc[s] * _rsqrt16(nacc[s]), 1e-8)
            sims[pl.ds(sim_base + s * L, L)] = acc[s] / en

    def chunk_body(c, _):
        loc_t = jnp.minimum(c * CT, n_t - CT)
        pltpu.sync_copy(
            emb_hbm.at[:, pl.ds((t0 + loc_t) * TILE, COLS_PER_CHUNK)], buf
        )

        def block_body(b, _):
            g0 = b * BG
            _dot_groups(buf, g0 * L, (loc_t * (TILE // L) + g0) * L, BG)
            return 0

        lax.fori_loop(0, GP_CHUNK // BG, block_body, 0)
        return 0

    lax.fori_loop(0, NCHUNK, chunk_body, 0)

    n_g = n_t * (TILE // L)
    @pl.when(wid == NW - 1)
    def _():
        pltpu.sync_copy(tail_hbm, tbuf)
        _dot_groups(tbuf, 0, n_g * L, REM // L)

    n_gt = n_g + jnp.where(wid == NW - 1, REM // L, 0)

    mask0 = iota == 0
    base_elem = t0 * TILE

    def select_body(j, _):
        def scan_body(g, ma):
            m, a = ma
            v = sims[pl.ds(g * L, L)]
            idxv = _splat_i(base_elem + g * L) + iota
            upd = v > m
            return jnp.where(upd, v, m), jnp.where(upd, idxv, a)

        m, a = lax.fori_loop(
            0, n_gt, scan_body, (_splat_f(NEG_INF), _splat_i(0))
        )
        mx = jnp.max(m)
        eq = m == _splat_f(mx)
        pos = jnp.min(jnp.where(eq, a, jnp.int32(2**30)))
        jv = _splat_i(j)
        plsc.store_scatter(cv, [jv], _splat_f(mx), mask=mask0)
        plsc.store_scatter(ci, [jv], _splat_i(pos), mask=mask0)
        plsc.store_scatter(
            sims, [_splat_i(pos - base_elem)], _splat_f(NEG_INF), mask=mask0
        )
        return 0

    lax.fori_loop(0, K, select_body, 0)

    pltpu.sync_copy(cv, cval_hbm.at[pl.ds(wid * K, K)])
    pltpu.sync_copy(ci, cidx_hbm.at[pl.ds(wid * K, K)])


@functools.partial(
    pl.kernel,
    out_type=(
        jax.ShapeDtypeStruct((K,), jnp.float32),
        jax.ShapeDtypeStruct((K,), jnp.int32),
    ),
    mesh=_MESH,
    scratch_types=[
        pltpu.VMEM((NW * K,), jnp.float32),
        pltpu.VMEM((NW * K,), jnp.int32),
        pltpu.VMEM((K,), jnp.float32),
        pltpu.VMEM((K,), jnp.int32),
        pltpu.VMEM((HID,), jnp.float32),
    ],
    compiler_params=_PARAMS,
)
def _merge(cval_hbm, cidx_hbm, q_hbm, score_hbm, idx_hbm,
           cvv, cii, selv, seli, qv):
    wid = lax.axis_index("s") * NC + lax.axis_index("c")
    iota = _iota16()
    mask0 = iota == 0

    @pl.when(wid == 0)
    def _():
        pltpu.sync_copy(cval_hbm, cvv)
        pltpu.sync_copy(cidx_hbm, cii)
        pltpu.sync_copy(q_hbm, qv)

        qsq = _splat_f(0.0)
        for t in range(HID // L):
            vq = qv[pl.ds(t * L, L)]
            qsq = qsq + vq * vq
        sv = _splat_f(jnp.sum(qsq))
        qn = jnp.maximum(sv * _rsqrt16(sv), 1e-8)

        def select_body(j, _):
            def scan_body(g, mae):
                m, a, e = mae
                v = cvv[pl.ds(g * L, L)]
                vi = cii[pl.ds(g * L, L)]
                idxv = _splat_i(g * L) + iota
                upd = v > m
                return (jnp.where(upd, v, m), jnp.where(upd, idxv, a),
                        jnp.where(upd, vi, e))

            m, a, e = lax.fori_loop(
                0, NW * K // L, scan_body,
                (_splat_f(NEG_INF), _splat_i(0), _splat_i(0)),
            )
            mx = jnp.max(m)
            eq = m == _splat_f(mx)
            pos = jnp.min(jnp.where(eq, a, jnp.int32(2**30)))
            posv = _splat_i(pos)
            epi_idx = jnp.min(jnp.where(a == posv, e, jnp.int32(2**30)))
            jv = _splat_i(j)
            plsc.store_scatter(selv, [jv], _splat_f(mx) / qn, mask=mask0)
            plsc.store_scatter(seli, [jv], _splat_i(epi_idx), mask=mask0)
            plsc.store_scatter(cvv, [posv], _splat_f(NEG_INF), mask=mask0)
            return 0

        lax.fori_loop(0, K, select_body, 0)

        pltpu.sync_copy(selv, score_hbm)
        pltpu.sync_copy(seli, idx_hbm)


def _gather_body(idx_ref, epi_ref, out_ref):
    i = pl.program_id(0)
    lane = idx_ref[i] % TILE
    blk = epi_ref[...]
    lanes = lax.broadcasted_iota(jnp.int32, (SEQ, HID, TILE), 2)
    out_ref[0] = jnp.sum(jnp.where(lanes == lane, blk, 0.0), axis=2)


def _gather_tc(idx, epi_t):
    grid_spec = pltpu.PrefetchScalarGridSpec(
        num_scalar_prefetch=1,
        grid=(K,),
        in_specs=[
            pl.BlockSpec(
                (SEQ, HID, TILE), lambda i, idx_ref: (0, 0, idx_ref[i] // TILE)
            )
        ],
        out_specs=pl.BlockSpec((1, SEQ, HID), lambda i, idx_ref: (i, 0, 0)),
    )
    return pl.pallas_call(
        _gather_body,
        grid_spec=grid_spec,
        out_shape=jax.ShapeDtypeStruct((K, SEQ, HID), jnp.float32),
    )(idx, epi_t)


def kernel(query, k, episodes, episode_embeddings):
    if query.ndim == 1:
        query = query[None, :]
    q0 = query[0]
    qb = jnp.repeat(q0, L)
    emb_t = episode_embeddings.T
    epi_t = jnp.transpose(episodes, (1, 2, 0))
    emb_tail = emb_t[:, NT_FULL * TILE:]
    cval, cidx = _partial_topk(emb_t, emb_tail, qb)
    scores, top_idx = _merge(cval, cidx, q0)
    retr = _gather_tc(top_idx, epi_t)
    scores = scores + jnp.asarray(k - k, dtype=scores.dtype)
    return retr, scores


# speedup vs baseline: 9.4165x; 1.0502x over previous
import functools

import jax
import jax.numpy as jnp
from jax import lax
from jax.experimental import pallas as pl
from jax.experimental.pallas import tpu as pltpu
from jax.experimental.pallas import tpu_sc as plsc

CAP = 100000
SEQ = 20
HID = 64
K = 32
L = 16
NC, NS = 2, 16
NW = NC * NS
TILE = 128
NT_FULL = CAP // TILE
REM = CAP - NT_FULL * TILE
CT = 5
COLS_PER_CHUNK = CT * TILE
NCHUNK = 5
BG = 10
GP_CHUNK = COLS_PER_CHUNK // L
MAXG_W = 25 * (TILE // L) + 2
NEG_INF = float("-inf")


def _iota16():
    return lax.iota(jnp.int32, L)


def _splat_f(x):
    return jnp.full((L,), x, dtype=jnp.float32)


def _splat_i(x):
    return jnp.full((L,), x, dtype=jnp.int32)


def _rsqrt16(x):
    i = plsc.bitcast(x, jnp.int32)
    i = jnp.int32(0x5F3759DF) - (i >> 1)
    r = plsc.bitcast(i, jnp.float32)
    for _ in range(3):
        r = r * (1.5 - 0.5 * x * r * r)
    return r


_MESH = plsc.VectorSubcoreMesh(core_axis_name="c", subcore_axis_name="s")
_PARAMS = pltpu.CompilerParams(needs_layout_passes=False)


@functools.partial(
    pl.kernel,
    out_type=(
        jax.ShapeDtypeStruct((NW * K,), jnp.float32),
        jax.ShapeDtypeStruct((NW * K,), jnp.int32),
    ),
    mesh=_MESH,
    scratch_types=[
        pltpu.VMEM((HID, COLS_PER_CHUNK), jnp.float32),
        pltpu.VMEM((HID, COLS_PER_CHUNK), jnp.float32),
        pltpu.VMEM((HID, REM), jnp.float32),
        pltpu.VMEM((MAXG_W * L,), jnp.float32),
        pltpu.VMEM((HID * L,), jnp.float32),
        pltpu.VMEM((K,), jnp.float32),
        pltpu.VMEM((K,), jnp.int32),
        pltpu.SemaphoreType.DMA,
        pltpu.SemaphoreType.DMA,
    ],
    compiler_params=_PARAMS,
)
def _partial_topk(emb_hbm, tail_hbm, q_hbm, cval_hbm, cidx_hbm,
                  buf0, buf1, tbuf, sims, qv, cv, ci, sem0, sem1):
    wid = lax.axis_index("s") * NC + lax.axis_index("c")
    t0 = (wid * NT_FULL) // NW
    n_t = ((wid + 1) * NT_FULL) // NW - t0
    iota = _iota16()

    pltpu.sync_copy(q_hbm, qv)

    def _dot_groups(src, col_base, sim_base, n):
        acc = [_splat_f(0.0) for _ in range(n)]
        nacc = [_splat_f(0.0) for _ in range(n)]
        for h in range(HID):
            qh = qv[pl.ds(h * L, L)]
            for s in range(n):
                v = src[h, pl.ds(col_base + s * L, L)]
                acc[s] = acc[s] + v * qh
                nacc[s] = nacc[s] + v * v
        for s in range(n):
            en = jnp.maximum(nacc[s] * _rsqrt16(nacc[s]), 1e-8)
            sims[pl.ds(sim_base + s * L, L)] = acc[s] / en

    def _loc_t(c):
        return jnp.minimum(c * CT, n_t - CT)

    def _start(c, dbuf, dsem):
        pltpu.async_copy(
            emb_hbm.at[:, pl.ds((t0 + _loc_t(c)) * TILE, COLS_PER_CHUNK)],
            dbuf, dsem,
        )

    def _drain(dbuf, dsem):
        pltpu.make_async_copy(
            emb_hbm.at[:, pl.ds(0, COLS_PER_CHUNK)], dbuf, dsem
        ).wait()

    def _compute(c, src):
        loc_t = _loc_t(c)

        def block_body(b, _):
            g0 = b * BG
            _dot_groups(src, g0 * L, (loc_t * (TILE // L) + g0) * L, BG)
            return 0

        lax.fori_loop(0, GP_CHUNK // BG, block_body, 0)

    _start(0, buf0, sem0)

    def chunk_body(c, _):
        @pl.when(c % 2 == 0)
        def _():
            _drain(buf0, sem0)

            @pl.when(c + 1 < NCHUNK)
            def _():
                _start(c + 1, buf1, sem1)

            _compute(c, buf0)

        @pl.when(c % 2 == 1)
        def _():
            _drain(buf1, sem1)

            @pl.when(c + 1 < NCHUNK)
            def _():
                _start(c + 1, buf0, sem0)

            _compute(c, buf1)

        return 0

    lax.fori_loop(0, NCHUNK, chunk_body, 0)

    n_g = n_t * (TILE // L)
    @pl.when(wid == NW - 1)
    def _():
        pltpu.sync_copy(tail_hbm, tbuf)
        _dot_groups(tbuf, 0, n_g * L, REM // L)

    n_gt = n_g + jnp.where(wid == NW - 1, REM // L, 0)

    mask0 = iota == 0
    base_elem = t0 * TILE

    def select_body(j, _):
        def scan_body(g, ma):
            m, a = ma
            v = sims[pl.ds(g * L, L)]
            idxv = _splat_i(base_elem + g * L) + iota
            upd = v > m
            return jnp.where(upd, v, m), jnp.where(upd, idxv, a)

        m, a = lax.fori_loop(
            0, n_gt, scan_body, (_splat_f(NEG_INF), _splat_i(0))
        )
        mx = jnp.max(m)
        eq = m == _splat_f(mx)
        pos = jnp.min(jnp.where(eq, a, jnp.int32(2**30)))
        jv = _splat_i(j)
        plsc.store_scatter(cv, [jv], _splat_f(mx), mask=mask0)
        plsc.store_scatter(ci, [jv], _splat_i(pos), mask=mask0)
        plsc.store_scatter(
            sims, [_splat_i(pos - base_elem)], _splat_f(NEG_INF), mask=mask0
        )
        return 0

    lax.fori_loop(0, K, select_body, 0)

    pltpu.sync_copy(cv, cval_hbm.at[pl.ds(wid * K, K)])
    pltpu.sync_copy(ci, cidx_hbm.at[pl.ds(wid * K, K)])


@functools.partial(
    pl.kernel,
    out_type=(
        jax.ShapeDtypeStruct((K,), jnp.float32),
        jax.ShapeDtypeStruct((K,), jnp.int32),
    ),
    mesh=_MESH,
    scratch_types=[
        pltpu.VMEM((NW * K,), jnp.float32),
        pltpu.VMEM((NW * K,), jnp.int32),
        pltpu.VMEM((K,), jnp.float32),
        pltpu.VMEM((K,), jnp.int32),
        pltpu.VMEM((HID,), jnp.float32),
    ],
    compiler_params=_PARAMS,
)
def _merge(cval_hbm, cidx_hbm, q_hbm, score_hbm, idx_hbm,
           cvv, cii, selv, seli, qv):
    wid = lax.axis_index("s") * NC + lax.axis_index("c")
    iota = _iota16()
    mask0 = iota == 0

    @pl.when(wid == 0)
    def _():
        pltpu.sync_copy(cval_hbm, cvv)
        pltpu.sync_copy(cidx_hbm, cii)
        pltpu.sync_copy(q_hbm, qv)

        qsq = _splat_f(0.0)
        for t in range(HID // L):
            vq = qv[pl.ds(t * L, L)]
            qsq = qsq + vq * vq
        sv = _splat_f(jnp.sum(qsq))
        qn = jnp.maximum(sv * _rsqrt16(sv), 1e-8)

        def select_body(j, _):
            def scan_body(g, mae):
                m, a, e = mae
                v = cvv[pl.ds(g * L, L)]
                vi = cii[pl.ds(g * L, L)]
                idxv = _splat_i(g * L) + iota
                upd = v > m
                return (jnp.where(upd, v, m), jnp.where(upd, idxv, a),
                        jnp.where(upd, vi, e))

            m, a, e = lax.fori_loop(
                0, NW * K // L, scan_body,
                (_splat_f(NEG_INF), _splat_i(0), _splat_i(0)),
            )
            mx = jnp.max(m)
            eq = m == _splat_f(mx)
            pos = jnp.min(jnp.where(eq, a, jnp.int32(2**30)))
            posv = _splat_i(pos)
            epi_idx = jnp.min(jnp.where(a == posv, e, jnp.int32(2**30)))
            jv = _splat_i(j)
            plsc.store_scatter(selv, [jv], _splat_f(mx) / qn, mask=mask0)
            plsc.store_scatter(seli, [jv], _splat_i(epi_idx), mask=mask0)
            plsc.store_scatter(cvv, [posv], _splat_f(NEG_INF), mask=mask0)
            return 0

        lax.fori_loop(0, K, select_body, 0)

        pltpu.sync_copy(selv, score_hbm)
        pltpu.sync_copy(seli, idx_hbm)


def _gather_body(idx_ref, epi_ref, out_ref):
    i = pl.program_id(0)
    lane = idx_ref[i] % TILE
    blk = epi_ref[...]
    lanes = lax.broadcasted_iota(jnp.int32, (SEQ, HID, TILE), 2)
    out_ref[0] = jnp.sum(jnp.where(lanes == lane, blk, 0.0), axis=2)


def _gather_tc(idx, epi_t):
    grid_spec = pltpu.PrefetchScalarGridSpec(
        num_scalar_prefetch=1,
        grid=(K,),
        in_specs=[
            pl.BlockSpec(
                (SEQ, HID, TILE), lambda i, idx_ref: (0, 0, idx_ref[i] // TILE)
            )
        ],
        out_specs=pl.BlockSpec((1, SEQ, HID), lambda i, idx_ref: (i, 0, 0)),
    )
    return pl.pallas_call(
        _gather_body,
        grid_spec=grid_spec,
        out_shape=jax.ShapeDtypeStruct((K, SEQ, HID), jnp.float32),
    )(idx, epi_t)


def kernel(query, k, episodes, episode_embeddings):
    if query.ndim == 1:
        query = query[None, :]
    q0 = query[0]
    qb = jnp.repeat(q0, L)
    emb_t = episode_embeddings.T
    epi_t = jnp.transpose(episodes, (1, 2, 0))
    emb_tail = emb_t[:, NT_FULL * TILE:]
    cval, cidx = _partial_topk(emb_t, emb_tail, qb)
    scores, top_idx = _merge(cval, cidx, q0)
    retr = _gather_tc(top_idx, epi_t)
    scores = scores + jnp.asarray(k - k, dtype=scores.dtype)
    return retr, scores


# speedup vs baseline: 11.0765x; 1.1763x over previous
import functools

import jax
import jax.numpy as jnp
from jax import lax
from jax.experimental import pallas as pl
from jax.experimental.pallas import tpu as pltpu
from jax.experimental.pallas import tpu_sc as plsc

CAP = 100000
SEQ = 20
HID = 64
K = 32
L = 16
NC, NS = 2, 16
NW = NC * NS
TILE = 128
NT_FULL = CAP // TILE
REM = CAP - NT_FULL * TILE
CT = 5
COLS_PER_CHUNK = CT * TILE
NCHUNK = 5
BG = 10
GP_CHUNK = COLS_PER_CHUNK // L
MAXG_W = 25 * (TILE // L) + 2
BB = 10
NBMAX = (MAXG_W + BB - 1) // BB
NEG_INF = float("-inf")
BIG_I = 2**30


def _iota16():
    return lax.iota(jnp.int32, L)


def _splat_f(x):
    return jnp.full((L,), x, dtype=jnp.float32)


def _splat_i(x):
    return jnp.full((L,), x, dtype=jnp.int32)


def _rsqrt16(x):
    i = plsc.bitcast(x, jnp.int32)
    i = jnp.int32(0x5F3759DF) - (i >> 1)
    r = plsc.bitcast(i, jnp.float32)
    for _ in range(3):
        r = r * (1.5 - 0.5 * x * r * r)
    return r


_MESH = plsc.VectorSubcoreMesh(core_axis_name="c", subcore_axis_name="s")
_PARAMS = pltpu.CompilerParams(needs_layout_passes=False)


@functools.partial(
    pl.kernel,
    out_type=(
        jax.ShapeDtypeStruct((NW * K,), jnp.float32),
        jax.ShapeDtypeStruct((NW * K,), jnp.int32),
    ),
    mesh=_MESH,
    scratch_types=[
        pltpu.VMEM((HID, COLS_PER_CHUNK), jnp.float32),
        pltpu.VMEM((HID, COLS_PER_CHUNK), jnp.float32),
        pltpu.VMEM((HID, REM), jnp.float32),
        pltpu.VMEM((MAXG_W * L,), jnp.float32),
        pltpu.VMEM((HID * L,), jnp.float32),
        pltpu.VMEM((K,), jnp.float32),
        pltpu.VMEM((K,), jnp.int32),
        pltpu.VMEM((NBMAX * L,), jnp.float32),
        pltpu.SemaphoreType.DMA,
        pltpu.SemaphoreType.DMA,
    ],
    compiler_params=_PARAMS,
)
def _partial_topk(emb_hbm, tail_hbm, q_hbm, cval_hbm, cidx_hbm,
                  buf0, buf1, tbuf, sims, qv, cv, ci, bm, sem0, sem1):
    wid = lax.axis_index("s") * NC + lax.axis_index("c")
    t0 = (wid * NT_FULL) // NW
    n_t = ((wid + 1) * NT_FULL) // NW - t0
    iota = _iota16()

    pltpu.sync_copy(q_hbm, qv)

    def _dot_groups(src, col_base, sim_base, n):
        acc = [_splat_f(0.0) for _ in range(n)]
        nacc = [_splat_f(0.0) for _ in range(n)]
        for h in range(HID):
            qh = qv[pl.ds(h * L, L)]
            for s in range(n):
                v = src[h, pl.ds(col_base + s * L, L)]
                acc[s] = acc[s] + v * qh
                nacc[s] = nacc[s] + v * v
        for s in range(n):
            en = jnp.maximum(nacc[s] * _rsqrt16(nacc[s]), 1e-8)
            sims[pl.ds(sim_base + s * L, L)] = acc[s] / en

    def _loc_t(c):
        return jnp.minimum(c * CT, n_t - CT)

    def _start(c, dbuf, dsem):
        pltpu.async_copy(
            emb_hbm.at[:, pl.ds((t0 + _loc_t(c)) * TILE, COLS_PER_CHUNK)],
            dbuf, dsem,
        )

    def _drain(dbuf, dsem):
        pltpu.make_async_copy(
            emb_hbm.at[:, pl.ds(0, COLS_PER_CHUNK)], dbuf, dsem
        ).wait()

    def _compute(c, src):
        loc_t = _loc_t(c)

        def block_body(b, _):
            g0 = b * BG
            _dot_groups(src, g0 * L, (loc_t * (TILE // L) + g0) * L, BG)
            return 0

        lax.fori_loop(0, GP_CHUNK // BG, block_body, 0)

    _start(0, buf0, sem0)

    def chunk_body(c, _):
        @pl.when(c % 2 == 0)
        def _():
            _drain(buf0, sem0)

            @pl.when(c + 1 < NCHUNK)
            def _():
                _start(c + 1, buf1, sem1)

            _compute(c, buf0)

        @pl.when(c % 2 == 1)
        def _():
            _drain(buf1, sem1)

            @pl.when(c + 1 < NCHUNK)
            def _():
                _start(c + 1, buf0, sem0)

            _compute(c, buf1)

        return 0

    lax.fori_loop(0, NCHUNK, chunk_body, 0)

    n_g = n_t * (TILE // L)
    @pl.when(wid == NW - 1)
    def _():
        pltpu.sync_copy(tail_hbm, tbuf)
        _dot_groups(tbuf, 0, n_g * L, REM // L)

    n_gt = n_g + jnp.where(wid == NW - 1, REM // L, 0)

    mask0 = iota == 0
    base_elem = t0 * TILE
    n_b = (n_gt + BB - 1) // BB

    def _block_max(glo, ghi):
        def b_scan(g, m):
            return jnp.maximum(m, sims[pl.ds(g * L, L)])
        return lax.fori_loop(glo, ghi, b_scan, _splat_f(NEG_INF))

    def pass1(nb, _):
        bm[pl.ds(nb * L, L)] = _block_max(
            nb * BB, jnp.minimum(nb * BB + BB, n_gt)
        )
        return 0

    lax.fori_loop(0, n_b, pass1, 0)

    def select_body(j, _):
        def bm_scan(nb, mb):
            m, b = mb
            v = bm[pl.ds(nb * L, L)]
            upd = v > m
            return jnp.where(upd, v, m), jnp.where(upd, _splat_i(nb), b)

        m, b = lax.fori_loop(
            0, n_b, bm_scan, (_splat_f(NEG_INF), _splat_i(0))
        )
        mx = jnp.max(m)
        mxv = _splat_f(mx)
        bsel = jnp.min(jnp.where(m == mxv, b, BIG_I))
        glo = bsel * BB
        ghi = jnp.minimum(glo + BB, n_gt)

        def pos_scan(g, pv):
            v = sims[pl.ds(g * L, L)]
            idxv = _splat_i(g * L) + iota
            return jnp.minimum(pv, jnp.where(v == mxv, idxv, BIG_I))

        pos = jnp.min(lax.fori_loop(glo, ghi, pos_scan, _splat_i(2**30)))
        jv = _splat_i(j)
        plsc.store_scatter(cv, [jv], mxv, mask=mask0)
        plsc.store_scatter(ci, [jv], _splat_i(base_elem + pos), mask=mask0)
        plsc.store_scatter(sims, [_splat_i(pos)], _splat_f(NEG_INF), mask=mask0)
        bm[pl.ds(bsel * L, L)] = _block_max(glo, ghi)
        return 0

    lax.fori_loop(0, K, select_body, 0)

    pltpu.sync_copy(cv, cval_hbm.at[pl.ds(wid * K, K)])
    pltpu.sync_copy(ci, cidx_hbm.at[pl.ds(wid * K, K)])


@functools.partial(
    pl.kernel,
    out_type=(
        jax.ShapeDtypeStruct((K,), jnp.float32),
        jax.ShapeDtypeStruct((K,), jnp.int32),
    ),
    mesh=_MESH,
    scratch_types=[
        pltpu.VMEM((NW * K,), jnp.float32),
        pltpu.VMEM((NW * K,), jnp.int32),
        pltpu.VMEM((K,), jnp.float32),
        pltpu.VMEM((K,), jnp.int32),
        pltpu.VMEM((HID,), jnp.float32),
    ],
    compiler_params=_PARAMS,
)
def _merge(cval_hbm, cidx_hbm, q_hbm, score_hbm, idx_hbm,
           cvv, cii, selv, seli, qv):
    wid = lax.axis_index("s") * NC + lax.axis_index("c")
    iota = _iota16()
    mask0 = iota == 0

    @pl.when(wid == 0)
    def _():
        pltpu.sync_copy(cval_hbm, cvv)
        pltpu.sync_copy(cidx_hbm, cii)
        pltpu.sync_copy(q_hbm, qv)

        qsq = _splat_f(0.0)
        for t in range(HID // L):
            vq = qv[pl.ds(t * L, L)]
            qsq = qsq + vq * vq
        sv = _splat_f(jnp.sum(qsq))
        qn = jnp.maximum(sv * _rsqrt16(sv), 1e-8)

        def select_body(j, _):
            def scan_body(g, mae):
                m, a, e = mae
                v = cvv[pl.ds(g * L, L)]
                vi = cii[pl.ds(g * L, L)]
                idxv = _splat_i(g * L) + iota
                upd = v > m
                return (jnp.where(upd, v, m), jnp.where(upd, idxv, a),
                        jnp.where(upd, vi, e))

            m, a, e = lax.fori_loop(
                0, NW * K // L, scan_body,
                (_splat_f(NEG_INF), _splat_i(0), _splat_i(0)),
            )
            mx = jnp.max(m)
            eq = m == _splat_f(mx)
            pos = jnp.min(jnp.where(eq, a, jnp.int32(2**30)))
            posv = _splat_i(pos)
            epi_idx = jnp.min(jnp.where(a == posv, e, jnp.int32(2**30)))
            jv = _splat_i(j)
            plsc.store_scatter(selv, [jv], _splat_f(mx) / qn, mask=mask0)
            plsc.store_scatter(seli, [jv], _splat_i(epi_idx), mask=mask0)
            plsc.store_scatter(cvv, [posv], _splat_f(NEG_INF), mask=mask0)
            return 0

        lax.fori_loop(0, K, select_body, 0)

        pltpu.sync_copy(selv, score_hbm)
        pltpu.sync_copy(seli, idx_hbm)


def _gather_body(idx_ref, epi_ref, out_ref):
    i = pl.program_id(0)
    lane = idx_ref[i] % TILE
    blk = epi_ref[...]
    lanes = lax.broadcasted_iota(jnp.int32, (SEQ, HID, TILE), 2)
    out_ref[0] = jnp.sum(jnp.where(lanes == lane, blk, 0.0), axis=2)


def _gather_tc(idx, epi_t):
    grid_spec = pltpu.PrefetchScalarGridSpec(
        num_scalar_prefetch=1,
        grid=(K,),
        in_specs=[
            pl.BlockSpec(
                (SEQ, HID, TILE), lambda i, idx_ref: (0, 0, idx_ref[i] // TILE)
            )
        ],
        out_specs=pl.BlockSpec((1, SEQ, HID), lambda i, idx_ref: (i, 0, 0)),
    )
    return pl.pallas_call(
        _gather_body,
        grid_spec=grid_spec,
        out_shape=jax.ShapeDtypeStruct((K, SEQ, HID), jnp.float32),
    )(idx, epi_t)


def kernel(query, k, episodes, episode_embeddings):
    if query.ndim == 1:
        query = query[None, :]
    q0 = query[0]
    qb = jnp.repeat(q0, L)
    emb_t = episode_embeddings.T
    epi_t = jnp.transpose(episodes, (1, 2, 0))
    emb_tail = emb_t[:, NT_FULL * TILE:]
    cval, cidx = _partial_topk(emb_t, emb_tail, qb)
    scores, top_idx = _merge(cval, cidx, q0)
    retr = _gather_tc(top_idx, epi_t)
    scores = scores + jnp.asarray(k - k, dtype=scores.dtype)
    return retr, scores


# speedup vs baseline: 12.1164x; 1.0939x over previous
import functools

import jax
import jax.numpy as jnp
from jax import lax
from jax.experimental import pallas as pl
from jax.experimental.pallas import tpu as pltpu
from jax.experimental.pallas import tpu_sc as plsc

CAP = 100000
SEQ = 20
HID = 64
K = 32
L = 16
NC, NS = 2, 16
NW = NC * NS
TILE = 128
NT_FULL = CAP // TILE
REM = CAP - NT_FULL * TILE
CT = 5
COLS_PER_CHUNK = CT * TILE
NCHUNK = 5
BG = 10
GP_CHUNK = COLS_PER_CHUNK // L
MAXG_W = 25 * (TILE // L) + 2
BB = 10
NBMAX = (MAXG_W + BB - 1) // BB
NEG_INF = float("-inf")
BIG_I = 2**30


def _iota16():
    return lax.iota(jnp.int32, L)


def _splat_f(x):
    return jnp.full((L,), x, dtype=jnp.float32)


def _splat_i(x):
    return jnp.full((L,), x, dtype=jnp.int32)


def _rsqrt16(x):
    i = plsc.bitcast(x, jnp.int32)
    i = jnp.int32(0x5F3759DF) - (i >> 1)
    r = plsc.bitcast(i, jnp.float32)
    for _ in range(3):
        r = r * (1.5 - 0.5 * x * r * r)
    return r


_MESH = plsc.VectorSubcoreMesh(core_axis_name="c", subcore_axis_name="s")
_PARAMS = pltpu.CompilerParams(needs_layout_passes=False)


@functools.partial(
    pl.kernel,
    out_type=(
        jax.ShapeDtypeStruct((NW * K,), jnp.float32),
        jax.ShapeDtypeStruct((NW * K,), jnp.int32),
    ),
    mesh=_MESH,
    scratch_types=[
        pltpu.VMEM((HID, COLS_PER_CHUNK), jnp.float32),
        pltpu.VMEM((HID, COLS_PER_CHUNK), jnp.float32),
        pltpu.VMEM((HID, REM), jnp.float32),
        pltpu.VMEM((MAXG_W * L,), jnp.float32),
        pltpu.VMEM((HID * L,), jnp.float32),
        pltpu.VMEM((K,), jnp.float32),
        pltpu.VMEM((K,), jnp.int32),
        pltpu.VMEM((NBMAX * L,), jnp.float32),
        pltpu.SemaphoreType.DMA,
        pltpu.SemaphoreType.DMA,
    ],
    compiler_params=_PARAMS,
)
def _partial_topk(emb_hbm, tail_hbm, q_hbm, cval_hbm, cidx_hbm,
                  buf0, buf1, tbuf, sims, qv, cv, ci, bm, sem0, sem1):
    wid = lax.axis_index("s") * NC + lax.axis_index("c")
    t0 = (wid * NT_FULL) // NW
    n_t = ((wid + 1) * NT_FULL) // NW - t0
    iota = _iota16()

    pltpu.sync_copy(q_hbm, qv)

    def _dot_groups(src, col_base, sim_base, n):
        acc = [_splat_f(0.0) for _ in range(n)]
        nacc = [_splat_f(0.0) for _ in range(n)]
        for h in range(HID):
            qh = qv[pl.ds(h * L, L)]
            for s in range(n):
                v = src[h, pl.ds(col_base + s * L, L)]
                acc[s] = acc[s] + v * qh
                nacc[s] = nacc[s] + v * v
        for s in range(n):
            en = jnp.maximum(nacc[s] * _rsqrt16(nacc[s]), 1e-8)
            sims[pl.ds(sim_base + s * L, L)] = acc[s] / en

    def _loc_t(c):
        return jnp.minimum(c * CT, n_t - CT)

    def _start(c, dbuf, dsem):
        pltpu.async_copy(
            emb_hbm.at[:, pl.ds((t0 + _loc_t(c)) * TILE, COLS_PER_CHUNK)],
            dbuf, dsem,
        )

    def _drain(dbuf, dsem):
        pltpu.make_async_copy(
            emb_hbm.at[:, pl.ds(0, COLS_PER_CHUNK)], dbuf, dsem
        ).wait()

    def _compute(c, src):
        loc_t = _loc_t(c)

        def block_body(b, _):
            g0 = b * BG
            _dot_groups(src, g0 * L, (loc_t * (TILE // L) + g0) * L, BG)
            return 0

        lax.fori_loop(0, GP_CHUNK // BG, block_body, 0)

    _start(0, buf0, sem0)

    def chunk_body(c, _):
        @pl.when(c % 2 == 0)
        def _():
            _drain(buf0, sem0)

            @pl.when(c + 1 < NCHUNK)
            def _():
                _start(c + 1, buf1, sem1)

            _compute(c, buf0)

        @pl.when(c % 2 == 1)
        def _():
            _drain(buf1, sem1)

            @pl.when(c + 1 < NCHUNK)
            def _():
                _start(c + 1, buf0, sem0)

            _compute(c, buf1)

        return 0

    lax.fori_loop(0, NCHUNK, chunk_body, 0)

    n_g = n_t * (TILE // L)
    @pl.when(wid == NW - 1)
    def _():
        pltpu.sync_copy(tail_hbm, tbuf)
        _dot_groups(tbuf, 0, n_g * L, REM // L)

    n_gt = n_g + jnp.where(wid == NW - 1, REM // L, 0)

    mask0 = iota == 0
    base_elem = t0 * TILE
    n_b = (n_gt + BB - 1) // BB

    def _block_max(glo, ghi):
        def b_scan(g, m):
            return jnp.maximum(m, sims[pl.ds(g * L, L)])
        return lax.fori_loop(glo, ghi, b_scan, _splat_f(NEG_INF))

    def pass1(nb, _):
        bm[pl.ds(nb * L, L)] = _block_max(
            nb * BB, jnp.minimum(nb * BB + BB, n_gt)
        )
        return 0

    lax.fori_loop(0, n_b, pass1, 0)

    def select_body(j, _):
        def bm_scan(nb, mb):
            m, b = mb
            v = bm[pl.ds(nb * L, L)]
            upd = v > m
            return jnp.where(upd, v, m), jnp.where(upd, _splat_i(nb), b)

        m, b = lax.fori_loop(
            0, n_b, bm_scan, (_splat_f(NEG_INF), _splat_i(0))
        )
        mx = jnp.max(m)
        mxv = _splat_f(mx)
        bsel = jnp.min(jnp.where(m == mxv, b, BIG_I))
        glo = bsel * BB
        ghi = jnp.minimum(glo + BB, n_gt)

        def pos_scan(g, pv):
            v = sims[pl.ds(g * L, L)]
            idxv = _splat_i(g * L) + iota
            return jnp.minimum(pv, jnp.where(v == mxv, idxv, BIG_I))

        pos = jnp.min(lax.fori_loop(glo, ghi, pos_scan, _splat_i(2**30)))
        jv = _splat_i(j)
        plsc.store_scatter(cv, [jv], mxv, mask=mask0)
        plsc.store_scatter(ci, [jv], _splat_i(base_elem + pos), mask=mask0)
        plsc.store_scatter(sims, [_splat_i(pos)], _splat_f(NEG_INF), mask=mask0)
        bm[pl.ds(bsel * L, L)] = _block_max(glo, ghi)
        return 0

    lax.fori_loop(0, K, select_body, 0)

    pltpu.sync_copy(cv, cval_hbm.at[pl.ds(wid * K, K)])
    pltpu.sync_copy(ci, cidx_hbm.at[pl.ds(wid * K, K)])


@functools.partial(
    pl.kernel,
    out_type=(
        jax.ShapeDtypeStruct((K,), jnp.float32),
        jax.ShapeDtypeStruct((K,), jnp.int32),
    ),
    mesh=_MESH,
    scratch_types=[
        pltpu.VMEM((NW * K,), jnp.float32),
        pltpu.VMEM((NW * K,), jnp.int32),
        pltpu.VMEM((K,), jnp.float32),
        pltpu.VMEM((K,), jnp.int32),
        pltpu.VMEM((HID,), jnp.float32),
        pltpu.VMEM((8 * L,), jnp.float32),
    ],
    compiler_params=_PARAMS,
)
def _merge(cval_hbm, cidx_hbm, q_hbm, score_hbm, idx_hbm,
           cvv, cii, selv, seli, qv, bm2):
    wid = lax.axis_index("s") * NC + lax.axis_index("c")
    iota = _iota16()
    mask0 = iota == 0

    @pl.when(wid == 0)
    def _():
        pltpu.sync_copy(cval_hbm, cvv)
        pltpu.sync_copy(cidx_hbm, cii)
        pltpu.sync_copy(q_hbm, qv)

        qsq = _splat_f(0.0)
        for t in range(HID // L):
            vq = qv[pl.ds(t * L, L)]
            qsq = qsq + vq * vq
        sv = _splat_f(jnp.sum(qsq))
        qn = jnp.maximum(sv * _rsqrt16(sv), 1e-8)

        def _bmax(blk):
            m = _splat_f(NEG_INF)
            for t in range(8):
                m = jnp.maximum(m, cvv[pl.ds((blk * 8 + t) * L, L)])
            return m

        for nb in range(8):
            bm2[pl.ds(nb * L, L)] = _bmax(nb)

        def select_body(j, _):
            m = _splat_f(NEG_INF)
            b = _splat_i(0)
            for nb in range(8):
                v = bm2[pl.ds(nb * L, L)]
                upd = v > m
                m = jnp.where(upd, v, m)
                b = jnp.where(upd, _splat_i(nb), b)
            mx = jnp.max(m)
            mxv = _splat_f(mx)
            bsel = jnp.min(jnp.where(m == mxv, b, BIG_I))
            pv = _splat_i(BIG_I)
            for t in range(8):
                g = bsel * 8 + t
                v = cvv[pl.ds(g * L, L)]
                idxv = _splat_i(g * L) + iota
                pv = jnp.minimum(pv, jnp.where(v == mxv, idxv, BIG_I))
            pos = jnp.min(pv)
            vi = cii[pl.ds(pos - (pos % L), L)]
            epi_idx = jnp.min(jnp.where(iota == _splat_i(pos % L), vi, BIG_I))
            jv = _splat_i(j)
            plsc.store_scatter(selv, [jv], mxv / qn, mask=mask0)
            plsc.store_scatter(seli, [jv], _splat_i(epi_idx), mask=mask0)
            plsc.store_scatter(cvv, [_splat_i(pos)], _splat_f(NEG_INF),
                               mask=mask0)
            bm2[pl.ds(bsel * L, L)] = _bmax(bsel)
            return 0

        lax.fori_loop(0, K, select_body, 0)

        pltpu.sync_copy(selv, score_hbm)
        pltpu.sync_copy(seli, idx_hbm)


def _gather_body(idx_ref, epi_ref, out_ref):
    i = pl.program_id(0)
    lane = idx_ref[i] % TILE
    blk = epi_ref[...]
    lanes = lax.broadcasted_iota(jnp.int32, (SEQ, HID, TILE), 2)
    out_ref[0] = jnp.sum(jnp.where(lanes == lane, blk, 0.0), axis=2)


def _gather_tc(idx, epi_t):
    grid_spec = pltpu.PrefetchScalarGridSpec(
        num_scalar_prefetch=1,
        grid=(K,),
        in_specs=[
            pl.BlockSpec(
                (SEQ, HID, TILE), lambda i, idx_ref: (0, 0, idx_ref[i] // TILE)
            )
        ],
        out_specs=pl.BlockSpec((1, SEQ, HID), lambda i, idx_ref: (i, 0, 0)),
    )
    return pl.pallas_call(
        _gather_body,
        grid_spec=grid_spec,
        out_shape=jax.ShapeDtypeStruct((K, SEQ, HID), jnp.float32),
    )(idx, epi_t)


def kernel(query, k, episodes, episode_embeddings):
    if query.ndim == 1:
        query = query[None, :]
    q0 = query[0]
    qb = jnp.repeat(q0, L)
    emb_t = episode_embeddings.T
    epi_t = jnp.transpose(episodes, (1, 2, 0))
    emb_tail = emb_t[:, NT_FULL * TILE:]
    cval, cidx = _partial_topk(emb_t, emb_tail, qb)
    scores, top_idx = _merge(cval, cidx, q0)
    retr = _gather_tc(top_idx, epi_t)
    scores = scores + jnp.asarray(k - k, dtype=scores.dtype)
    return retr, scores
